# baseline (speedup 1.0000x reference)
import functools

import jax
import jax.numpy as jnp
from jax import lax
from jax.experimental import pallas as pl
from jax.experimental.pallas import tpu as pltpu
from jax.experimental.pallas import tpu_sc as plsc

N = 10000
E = 320000
D = 128
L = 4096

_NC = 2
_NS = 16
_HD = D // _NC
_CH = 125
_EPT = E // _NS
_NCHUNK = _EPT // _CH
_NBUF = 4
_GAHEAD = 2
_RPT = 640
_NPAD = _RPT * _NS


def _sc_scatter_add(hL, hR, src3, dst3, zeros):
    mesh = plsc.VectorSubcoreMesh(core_axis_name="c", subcore_axis_name="s")

    @functools.partial(
        pl.kernel,
        mesh=mesh,
        compiler_params=pltpu.CompilerParams(use_tc_tiling_on_sc=False),
        out_type=jax.ShapeDtypeStruct((_NC, _NPAD, _HD), jnp.float32),
        scratch_types=[
            pltpu.VMEM((_NCHUNK, _CH), jnp.int32),
            pltpu.VMEM((_NCHUNK, _CH), jnp.int32),
            pltpu.VMEM((_NBUF, _CH, _HD), jnp.float32),
            pltpu.VMEM_SHARED((_NPAD, _HD), jnp.float32),
            [pltpu.SemaphoreType.DMA] * _NBUF,
            [pltpu.SemaphoreType.DMA] * _NBUF,
        ],
    )
    def k(hL_hbm, hR_hbm, src_hbm, dst_hbm, z_hbm, out_hbm,
          src_v, dst_v, rows_v, acc, gsems, ssems):
        c = lax.axis_index("c")
        s = lax.axis_index("s")
        pltpu.sync_copy(z_hbm, acc.at[pl.ds(s * _RPT, _RPT)])
        pltpu.sync_copy(src_hbm.at[pl.ds(s * _NCHUNK, _NCHUNK)], src_v)
        pltpu.sync_copy(dst_hbm.at[pl.ds(s * _NCHUNK, _NCHUNK)], dst_v)
        plsc.subcore_barrier()

        def run(h_hbm):
            def g_start(j, b):
                pltpu.make_async_copy(
                    h_hbm.at[src_v.at[j]], rows_v.at[b], gsems[b]).start()

            def g_wait(j, b):
                pltpu.make_async_copy(
                    h_hbm.at[src_v.at[j]], rows_v.at[b], gsems[b]).wait()

            def s_start(j, b):
                pltpu.make_async_copy(
                    rows_v.at[b], acc.at[dst_v.at[j]], ssems[b]).start(add=True)

            def s_wait(j, b):
                pltpu.make_async_copy(
                    rows_v.at[b], acc.at[dst_v.at[j]], ssems[b]).wait()

            for b in range(_GAHEAD):
                g_start(b, b)

            def body(i, carry):
                j0 = i * _NBUF
                for t in range(_NBUF):
                    j = j0 + t
                    g_wait(j, t)
                    s_start(j, t)
                    nj = j + _GAHEAD
                    nb = (t + _GAHEAD) % _NBUF

                    @pl.when(nj < _NCHUNK)
                    def _(nj=nj, nb=nb):
                        @pl.when(nj - _NBUF >= 0)
                        def _():
                            s_wait(nj - _NBUF, nb)

                        g_start(nj, nb)

                return carry

            lax.fori_loop(0, _NCHUNK // _NBUF, body, 0)
            for t in range(_NBUF):
                s_wait(_NCHUNK - _NBUF + t, t)

        @pl.when(c == 0)
        def _():
            run(hL_hbm)

        @pl.when(c == 1)
        def _():
            run(hR_hbm)

        plsc.subcore_barrier()
        pltpu.sync_copy(acc.at[pl.ds(s * _RPT, _RPT)],
                        out_hbm.at[c, pl.ds(s * _RPT, _RPT)])

    return k(hL, hR, src3, dst3, zeros)


_BR = 1000


def _gin_dense(h, p, eps, W, b, Rw, Rb):
    def body(eps_ref, h_ref, p0_ref, p1_ref, W_ref, b_ref, Rw_ref, Rb_ref,
             o_ref, oL_ref, oR_ref):
        agg = jnp.concatenate([p0_ref[...], p1_ref[...]], axis=1)
        x = (1.0 + eps_ref[0]) * h_ref[...] + agg
        y = jnp.dot(x, W_ref[...], preferred_element_type=jnp.float32) + b_ref[...]
        z = jnp.maximum(y, 0.0)
        w = jnp.dot(z, Rw_ref[...], preferred_element_type=jnp.float32) + Rb_ref[...]
        res = z + jnp.maximum(w, 0.0)
        o_ref[...] = res
        oL_ref[...] = res[:, :_HD]
        oR_ref[...] = res[:, _HD:]

    return pl.pallas_call(
        body,
        grid=(N // _BR,),
        in_specs=[
            pl.BlockSpec(memory_space=pltpu.SMEM),
            pl.BlockSpec((_BR, D), lambda i: (i, 0)),
            pl.BlockSpec((_BR, _HD), lambda i: (i, 0)),
            pl.BlockSpec((_BR, _HD), lambda i: (i, 0)),
            pl.BlockSpec((D, D), lambda i: (0, 0)),
            pl.BlockSpec((1, D), lambda i: (0, 0)),
            pl.BlockSpec((D, D), lambda i: (0, 0)),
            pl.BlockSpec((1, D), lambda i: (0, 0)),
        ],
        out_specs=[
            pl.BlockSpec((_BR, D), lambda i: (i, 0)),
            pl.BlockSpec((_BR, _HD), lambda i: (i, 0)),
            pl.BlockSpec((_BR, _HD), lambda i: (i, 0)),
        ],
        out_shape=[
            jax.ShapeDtypeStruct((N, D), jnp.float32),
            jax.ShapeDtypeStruct((N, _HD), jnp.float32),
            jax.ShapeDtypeStruct((N, _HD), jnp.float32),
        ],
        compiler_params=pltpu.CompilerParams(dimension_semantics=("parallel",)),
    )(eps.reshape(1), h, p[0], p[1], W, b.reshape(1, D), Rw, Rb.reshape(1, D))


def _final(h1, p, eps, W, b, Rw, Rb, Lw, Lb):
    def body(eps_ref, h_ref, p0_ref, p1_ref, W_ref, b_ref, Rw_ref, Rb_ref,
             Lw_ref, Lb_ref, o_ref):
        h1b = h_ref[...]
        agg = jnp.concatenate([p0_ref[...], p1_ref[...]], axis=1)
        x = (1.0 + eps_ref[0]) * h1b + agg
        y = jnp.dot(x, W_ref[...], preferred_element_type=jnp.float32) + b_ref[...]
        z = jnp.maximum(y, 0.0)
        w = jnp.dot(z, Rw_ref[...], preferred_element_type=jnp.float32) + Rb_ref[...]
        h2b = z + jnp.maximum(w, 0.0)
        m = 0.5 * (h1b + h2b)
        o_ref[...] = jnp.dot(m, Lw_ref[...], preferred_element_type=jnp.float32) + Lb_ref[...]

    return pl.pallas_call(
        body,
        grid=(N // _BR,),
        in_specs=[
            pl.BlockSpec(memory_space=pltpu.SMEM),
            pl.BlockSpec((_BR, D), lambda i: (i, 0)),
            pl.BlockSpec((_BR, _HD), lambda i: (i, 0)),
            pl.BlockSpec((_BR, _HD), lambda i: (i, 0)),
            pl.BlockSpec((D, D), lambda i: (0, 0)),
            pl.BlockSpec((1, D), lambda i: (0, 0)),
            pl.BlockSpec((D, D), lambda i: (0, 0)),
            pl.BlockSpec((1, D), lambda i: (0, 0)),
            pl.BlockSpec((D, L), lambda i: (0, 0)),
            pl.BlockSpec((1, L), lambda i: (0, 0)),
        ],
        out_specs=pl.BlockSpec((_BR, L), lambda i: (i, 0)),
        out_shape=jax.ShapeDtypeStruct((N, L), jnp.float32),
        compiler_params=pltpu.CompilerParams(dimension_semantics=("parallel",)),
    )(eps.reshape(1), h1, p[0], p[1], W, b.reshape(1, D), Rw, Rb.reshape(1, D),
      Lw, Lb.reshape(1, L))


def kernel(inputs, edge_index, eps1, W1, b1, eps2, W2, b2, Rw1, Rb1, Rw2, Rb2, Lw, Lb):
    src3 = edge_index[0].reshape(E // _CH, _CH)
    dst3 = edge_index[1].reshape(E // _CH, _CH)
    zeros = jnp.zeros((_RPT, _HD), jnp.float32)
    p1 = _sc_scatter_add(inputs[:, :_HD], inputs[:, _HD:], src3, dst3, zeros)
    h1, h1L, h1R = _gin_dense(inputs, p1, eps1, W1, b1, Rw1, Rb1)
    p2 = _sc_scatter_add(h1L, h1R, src3, dst3, zeros)
    return _final(h1, p2, eps2, W2, b2, Rw2, Rb2, Lw, Lb)


# speedup vs baseline: 1.1401x; 1.1401x over previous
import functools

import jax
import jax.numpy as jnp
from jax import lax
from jax.experimental import pallas as pl
from jax.experimental.pallas import tpu as pltpu
from jax.experimental.pallas import tpu_sc as plsc

N = 10000
E = 320000
D = 128
L = 4096

_NC = 2
_NS = 16
_HD = D // _NC
_CH = 125
_EPT = E // _NS
_NCHUNK = _EPT // _CH
_NBUF = 4
_GAHEAD = 4
_RPT = 640
_NPAD = _RPT * _NS


def _sc_scatter_add(hL, hR, src3, dst3, zeros):
    mesh = plsc.VectorSubcoreMesh(core_axis_name="c", subcore_axis_name="s")

    @functools.partial(
        pl.kernel,
        mesh=mesh,
        compiler_params=pltpu.CompilerParams(use_tc_tiling_on_sc=False),
        out_type=jax.ShapeDtypeStruct((_NC, _NPAD, _HD), jnp.float32),
        scratch_types=[
            pltpu.VMEM((_NCHUNK, _CH), jnp.int32),
            pltpu.VMEM((_NCHUNK, _CH), jnp.int32),
            pltpu.VMEM((_NBUF, _CH, _HD), jnp.float32),
            pltpu.VMEM_SHARED((_NPAD, _HD), jnp.float32),
            [pltpu.SemaphoreType.DMA] * _NBUF,
            [pltpu.SemaphoreType.DMA] * _NBUF,
        ],
    )
    def k(hL_hbm, hR_hbm, src_hbm, dst_hbm, z_hbm, out_hbm,
          src_v, dst_v, rows_v, acc, gsems, ssems):
        c = lax.axis_index("c")
        s = lax.axis_index("s")
        pltpu.sync_copy(z_hbm, acc.at[pl.ds(s * _RPT, _RPT)])
        pltpu.sync_copy(src_hbm.at[pl.ds(s * _NCHUNK, _NCHUNK)], src_v)
        pltpu.sync_copy(dst_hbm.at[pl.ds(s * _NCHUNK, _NCHUNK)], dst_v)
        plsc.subcore_barrier()

        def run(h_hbm):
            def g_start(j, b):
                pltpu.make_async_copy(
                    h_hbm.at[src_v.at[j]], rows_v.at[b], gsems[b]).start()

            def g_wait(j, b):
                pltpu.make_async_copy(
                    h_hbm.at[src_v.at[j]], rows_v.at[b], gsems[b]).wait()

            def s_start(j, b):
                pltpu.make_async_copy(
                    rows_v.at[b], acc.at[dst_v.at[j]], ssems[b]).start(add=True)

            def s_wait(j, b):
                pltpu.make_async_copy(
                    rows_v.at[b], acc.at[dst_v.at[j]], ssems[b]).wait()

            for b in range(_GAHEAD):
                g_start(b, b)

            def body(i, carry):
                j0 = i * _NBUF
                for t in range(_NBUF):
                    j = j0 + t
                    g_wait(j, t)
                    pltpu.sync_copy(rows_v.at[t], acc.at[dst_v.at[j]], add=True)

                    @pl.when(j + _NBUF < _NCHUNK)
                    def _(j=j, t=t):
                        g_start(j + _NBUF, t)

                return carry

            lax.fori_loop(0, _NCHUNK // _NBUF, body, 0)

        @pl.when(c == 0)
        def _():
            run(hL_hbm)

        @pl.when(c == 1)
        def _():
            run(hR_hbm)

        plsc.subcore_barrier()
        pltpu.sync_copy(acc.at[pl.ds(s * _RPT, _RPT)],
                        out_hbm.at[c, pl.ds(s * _RPT, _RPT)])

    return k(hL, hR, src3, dst3, zeros)


_BR = 1000


def _gin_dense(h, p, eps, W, b, Rw, Rb):
    def body(eps_ref, h_ref, p0_ref, p1_ref, W_ref, b_ref, Rw_ref, Rb_ref,
             o_ref, oL_ref, oR_ref):
        agg = jnp.concatenate([p0_ref[...], p1_ref[...]], axis=1)
        x = (1.0 + eps_ref[0]) * h_ref[...] + agg
        y = jnp.dot(x, W_ref[...], preferred_element_type=jnp.float32) + b_ref[...]
        z = jnp.maximum(y, 0.0)
        w = jnp.dot(z, Rw_ref[...], preferred_element_type=jnp.float32) + Rb_ref[...]
        res = z + jnp.maximum(w, 0.0)
        o_ref[...] = res
        oL_ref[...] = res[:, :_HD]
        oR_ref[...] = res[:, _HD:]

    return pl.pallas_call(
        body,
        grid=(N // _BR,),
        in_specs=[
            pl.BlockSpec(memory_space=pltpu.SMEM),
            pl.BlockSpec((_BR, D), lambda i: (i, 0)),
            pl.BlockSpec((_BR, _HD), lambda i: (i, 0)),
            pl.BlockSpec((_BR, _HD), lambda i: (i, 0)),
            pl.BlockSpec((D, D), lambda i: (0, 0)),
            pl.BlockSpec((1, D), lambda i: (0, 0)),
            pl.BlockSpec((D, D), lambda i: (0, 0)),
            pl.BlockSpec((1, D), lambda i: (0, 0)),
        ],
        out_specs=[
            pl.BlockSpec((_BR, D), lambda i: (i, 0)),
            pl.BlockSpec((_BR, _HD), lambda i: (i, 0)),
            pl.BlockSpec((_BR, _HD), lambda i: (i, 0)),
        ],
        out_shape=[
            jax.ShapeDtypeStruct((N, D), jnp.float32),
            jax.ShapeDtypeStruct((N, _HD), jnp.float32),
            jax.ShapeDtypeStruct((N, _HD), jnp.float32),
        ],
        compiler_params=pltpu.CompilerParams(dimension_semantics=("parallel",)),
    )(eps.reshape(1), h, p[0], p[1], W, b.reshape(1, D), Rw, Rb.reshape(1, D))


def _final(h1, p, eps, W, b, Rw, Rb, Lw, Lb):
    def body(eps_ref, h_ref, p0_ref, p1_ref, W_ref, b_ref, Rw_ref, Rb_ref,
             Lw_ref, Lb_ref, o_ref):
        h1b = h_ref[...]
        agg = jnp.concatenate([p0_ref[...], p1_ref[...]], axis=1)
        x = (1.0 + eps_ref[0]) * h1b + agg
        y = jnp.dot(x, W_ref[...], preferred_element_type=jnp.float32) + b_ref[...]
        z = jnp.maximum(y, 0.0)
        w = jnp.dot(z, Rw_ref[...], preferred_element_type=jnp.float32) + Rb_ref[...]
        h2b = z + jnp.maximum(w, 0.0)
        m = 0.5 * (h1b + h2b)
        o_ref[...] = jnp.dot(m, Lw_ref[...], preferred_element_type=jnp.float32) + Lb_ref[...]

    return pl.pallas_call(
        body,
        grid=(N // _BR,),
        in_specs=[
            pl.BlockSpec(memory_space=pltpu.SMEM),
            pl.BlockSpec((_BR, D), lambda i: (i, 0)),
            pl.BlockSpec((_BR, _HD), lambda i: (i, 0)),
            pl.BlockSpec((_BR, _HD), lambda i: (i, 0)),
            pl.BlockSpec((D, D), lambda i: (0, 0)),
            pl.BlockSpec((1, D), lambda i: (0, 0)),
            pl.BlockSpec((D, D), lambda i: (0, 0)),
            pl.BlockSpec((1, D), lambda i: (0, 0)),
            pl.BlockSpec((D, L), lambda i: (0, 0)),
            pl.BlockSpec((1, L), lambda i: (0, 0)),
        ],
        out_specs=pl.BlockSpec((_BR, L), lambda i: (i, 0)),
        out_shape=jax.ShapeDtypeStruct((N, L), jnp.float32),
        compiler_params=pltpu.CompilerParams(dimension_semantics=("parallel",)),
    )(eps.reshape(1), h1, p[0], p[1], W, b.reshape(1, D), Rw, Rb.reshape(1, D),
      Lw, Lb.reshape(1, L))


def kernel(inputs, edge_index, eps1, W1, b1, eps2, W2, b2, Rw1, Rb1, Rw2, Rb2, Lw, Lb):
    src3 = edge_index[0].reshape(E // _CH, _CH)
    dst3 = edge_index[1].reshape(E // _CH, _CH)
    zeros = jnp.zeros((_RPT, _HD), jnp.float32)
    p1 = _sc_scatter_add(inputs[:, :_HD], inputs[:, _HD:], src3, dst3, zeros)
    h1, h1L, h1R = _gin_dense(inputs, p1, eps1, W1, b1, Rw1, Rb1)
    p2 = _sc_scatter_add(h1L, h1R, src3, dst3, zeros)
    return _final(h1, p2, eps2, W2, b2, Rw2, Rb2, Lw, Lb)


# speedup vs baseline: 1.1403x; 1.0002x over previous
import functools

import jax
import jax.numpy as jnp
from jax import lax
from jax.experimental import pallas as pl
from jax.experimental.pallas import tpu as pltpu
from jax.experimental.pallas import tpu_sc as plsc

N = 10000
E = 320000
D = 128
L = 4096

_NC = 2
_NS = 16
_HD = D // _NC
_CH = 125
_EPT = E // _NS
_NCHUNK = _EPT // _CH
_NBUF = 5
_GAHEAD = 5
_RPT = 640
_NPAD = _RPT * _NS


def _sc_scatter_add(hL, hR, src3, dst3, zeros):
    mesh = plsc.VectorSubcoreMesh(core_axis_name="c", subcore_axis_name="s")

    @functools.partial(
        pl.kernel,
        mesh=mesh,
        compiler_params=pltpu.CompilerParams(use_tc_tiling_on_sc=False),
        out_type=jax.ShapeDtypeStruct((_NC, _NPAD, _HD), jnp.float32),
        scratch_types=[
            pltpu.VMEM((_NCHUNK, _CH), jnp.int32),
            pltpu.VMEM((_NCHUNK, _CH), jnp.int32),
            pltpu.VMEM((_NBUF, _CH, _HD), jnp.float32),
            pltpu.VMEM_SHARED((_NPAD, _HD), jnp.float32),
            [pltpu.SemaphoreType.DMA] * _NBUF,
            [pltpu.SemaphoreType.DMA] * _NBUF,
        ],
    )
    def k(hL_hbm, hR_hbm, src_hbm, dst_hbm, z_hbm, out_hbm,
          src_v, dst_v, rows_v, acc, gsems, ssems):
        c = lax.axis_index("c")
        s = lax.axis_index("s")
        pltpu.sync_copy(z_hbm, acc.at[pl.ds(s * _RPT, _RPT)])
        pltpu.sync_copy(src_hbm.at[pl.ds(s * _NCHUNK, _NCHUNK)], src_v)
        pltpu.sync_copy(dst_hbm.at[pl.ds(s * _NCHUNK, _NCHUNK)], dst_v)
        plsc.subcore_barrier()

        def run(h_hbm):
            def g_start(j, b):
                pltpu.make_async_copy(
                    h_hbm.at[src_v.at[j]], rows_v.at[b], gsems[b]).start()

            def g_wait(j, b):
                pltpu.make_async_copy(
                    h_hbm.at[src_v.at[j]], rows_v.at[b], gsems[b]).wait()

            def s_start(j, b):
                pltpu.make_async_copy(
                    rows_v.at[b], acc.at[dst_v.at[j]], ssems[b]).start(add=True)

            def s_wait(j, b):
                pltpu.make_async_copy(
                    rows_v.at[b], acc.at[dst_v.at[j]], ssems[b]).wait()

            for b in range(_GAHEAD):
                g_start(b, b)

            def body(i, carry):
                j0 = i * _NBUF
                for t in range(_NBUF):
                    j = j0 + t
                    g_wait(j, t)
                    pltpu.sync_copy(rows_v.at[t], acc.at[dst_v.at[j]], add=True)

                    @pl.when(j + _NBUF < _NCHUNK)
                    def _(j=j, t=t):
                        g_start(j + _NBUF, t)

                return carry

            lax.fori_loop(0, _NCHUNK // _NBUF, body, 0)

        @pl.when(c == 0)
        def _():
            run(hL_hbm)

        @pl.when(c == 1)
        def _():
            run(hR_hbm)

        plsc.subcore_barrier()
        pltpu.sync_copy(acc.at[pl.ds(s * _RPT, _RPT)],
                        out_hbm.at[c, pl.ds(s * _RPT, _RPT)])

    return k(hL, hR, src3, dst3, zeros)


_BR = 1000


def _gin_dense(h, p, eps, W, b, Rw, Rb):
    def body(eps_ref, h_ref, p0_ref, p1_ref, W_ref, b_ref, Rw_ref, Rb_ref,
             o_ref, oL_ref, oR_ref):
        agg = jnp.concatenate([p0_ref[...], p1_ref[...]], axis=1)
        x = (1.0 + eps_ref[0]) * h_ref[...] + agg
        y = jnp.dot(x, W_ref[...], preferred_element_type=jnp.float32) + b_ref[...]
        z = jnp.maximum(y, 0.0)
        w = jnp.dot(z, Rw_ref[...], preferred_element_type=jnp.float32) + Rb_ref[...]
        res = z + jnp.maximum(w, 0.0)
        o_ref[...] = res
        oL_ref[...] = res[:, :_HD]
        oR_ref[...] = res[:, _HD:]

    return pl.pallas_call(
        body,
        grid=(N // _BR,),
        in_specs=[
            pl.BlockSpec(memory_space=pltpu.SMEM),
            pl.BlockSpec((_BR, D), lambda i: (i, 0)),
            pl.BlockSpec((_BR, _HD), lambda i: (i, 0)),
            pl.BlockSpec((_BR, _HD), lambda i: (i, 0)),
            pl.BlockSpec((D, D), lambda i: (0, 0)),
            pl.BlockSpec((1, D), lambda i: (0, 0)),
            pl.BlockSpec((D, D), lambda i: (0, 0)),
            pl.BlockSpec((1, D), lambda i: (0, 0)),
        ],
        out_specs=[
            pl.BlockSpec((_BR, D), lambda i: (i, 0)),
            pl.BlockSpec((_BR, _HD), lambda i: (i, 0)),
            pl.BlockSpec((_BR, _HD), lambda i: (i, 0)),
        ],
        out_shape=[
            jax.ShapeDtypeStruct((N, D), jnp.float32),
            jax.ShapeDtypeStruct((N, _HD), jnp.float32),
            jax.ShapeDtypeStruct((N, _HD), jnp.float32),
        ],
        compiler_params=pltpu.CompilerParams(dimension_semantics=("parallel",)),
    )(eps.reshape(1), h, p[0], p[1], W, b.reshape(1, D), Rw, Rb.reshape(1, D))


def _final(h1, p, eps, W, b, Rw, Rb, Lw, Lb):
    def body(eps_ref, h_ref, p0_ref, p1_ref, W_ref, b_ref, Rw_ref, Rb_ref,
             Lw_ref, Lb_ref, o_ref):
        h1b = h_ref[...]
        agg = jnp.concatenate([p0_ref[...], p1_ref[...]], axis=1)
        x = (1.0 + eps_ref[0]) * h1b + agg
        y = jnp.dot(x, W_ref[...], preferred_element_type=jnp.float32) + b_ref[...]
        z = jnp.maximum(y, 0.0)
        w = jnp.dot(z, Rw_ref[...], preferred_element_type=jnp.float32) + Rb_ref[...]
        h2b = z + jnp.maximum(w, 0.0)
        m = 0.5 * (h1b + h2b)
        o_ref[...] = jnp.dot(m, Lw_ref[...], preferred_element_type=jnp.float32) + Lb_ref[...]

    return pl.pallas_call(
        body,
        grid=(N // _BR,),
        in_specs=[
            pl.BlockSpec(memory_space=pltpu.SMEM),
            pl.BlockSpec((_BR, D), lambda i: (i, 0)),
            pl.BlockSpec((_BR, _HD), lambda i: (i, 0)),
            pl.BlockSpec((_BR, _HD), lambda i: (i, 0)),
            pl.BlockSpec((D, D), lambda i: (0, 0)),
            pl.BlockSpec((1, D), lambda i: (0, 0)),
            pl.BlockSpec((D, D), lambda i: (0, 0)),
            pl.BlockSpec((1, D), lambda i: (0, 0)),
            pl.BlockSpec((D, L), lambda i: (0, 0)),
            pl.BlockSpec((1, L), lambda i: (0, 0)),
        ],
        out_specs=pl.BlockSpec((_BR, L), lambda i: (i, 0)),
        out_shape=jax.ShapeDtypeStruct((N, L), jnp.float32),
        compiler_params=pltpu.CompilerParams(dimension_semantics=("parallel",)),
    )(eps.reshape(1), h1, p[0], p[1], W, b.reshape(1, D), Rw, Rb.reshape(1, D),
      Lw, Lb.reshape(1, L))


def kernel(inputs, edge_index, eps1, W1, b1, eps2, W2, b2, Rw1, Rb1, Rw2, Rb2, Lw, Lb):
    src3 = edge_index[0].reshape(E // _CH, _CH)
    dst3 = edge_index[1].reshape(E // _CH, _CH)
    zeros = jnp.zeros((_RPT, _HD), jnp.float32)
    p1 = _sc_scatter_add(inputs[:, :_HD], inputs[:, _HD:], src3, dst3, zeros)
    h1, h1L, h1R = _gin_dense(inputs, p1, eps1, W1, b1, Rw1, Rb1)
    p2 = _sc_scatter_add(h1L, h1R, src3, dst3, zeros)
    return _final(h1, p2, eps2, W2, b2, Rw2, Rb2, Lw, Lb)


# speedup vs baseline: 1.2409x; 1.0883x over previous
import functools

import jax
import jax.numpy as jnp
from jax import lax
from jax.experimental import pallas as pl
from jax.experimental.pallas import tpu as pltpu
from jax.experimental.pallas import tpu_sc as plsc

N = 10000
E = 320000
D = 128
L = 4096

_NC = 2
_NS = 16
_HD = D // _NC
_CH = 125
_EPT = E // _NS
_NCHUNK = _EPT // _CH
_NBUF = 5
_GAHEAD = 5
_RPT = 640
_NPAD = _RPT * _NS


def _sc_scatter_add(hL, hR, eidx, zeros):
    mesh = plsc.VectorSubcoreMesh(core_axis_name="c", subcore_axis_name="s")

    @functools.partial(
        pl.kernel,
        mesh=mesh,
        compiler_params=pltpu.CompilerParams(use_tc_tiling_on_sc=False),
        out_type=jax.ShapeDtypeStruct((_NC, _NPAD, _HD), jnp.float32),
        scratch_types=[
            pltpu.VMEM((_NCHUNK, _CH), jnp.int32),
            pltpu.VMEM((_NCHUNK, _CH), jnp.int32),
            pltpu.VMEM((_NBUF, _CH, _HD), jnp.float32),
            pltpu.VMEM_SHARED((_NPAD, _HD), jnp.float32),
            [pltpu.SemaphoreType.DMA] * _NBUF,
            [pltpu.SemaphoreType.DMA] * _NBUF,
        ],
    )
    def k(hL_hbm, hR_hbm, eidx_hbm, z_hbm, out_hbm,
          src_v, dst_v, rows_v, acc, gsems, ssems):
        c = lax.axis_index("c")
        s = lax.axis_index("s")
        pltpu.sync_copy(z_hbm, acc.at[pl.ds(s * _RPT, _RPT)])
        pltpu.sync_copy(eidx_hbm.at[0, pl.ds(s * _NCHUNK, _NCHUNK)], src_v)
        pltpu.sync_copy(eidx_hbm.at[1, pl.ds(s * _NCHUNK, _NCHUNK)], dst_v)
        plsc.subcore_barrier()

        def run(h_hbm):
            def g_start(j, b):
                pltpu.make_async_copy(
                    h_hbm.at[src_v.at[j]], rows_v.at[b], gsems[b]).start()

            def g_wait(j, b):
                pltpu.make_async_copy(
                    h_hbm.at[src_v.at[j]], rows_v.at[b], gsems[b]).wait()

            def s_start(j, b):
                pltpu.make_async_copy(
                    rows_v.at[b], acc.at[dst_v.at[j]], ssems[b]).start(add=True)

            def s_wait(j, b):
                pltpu.make_async_copy(
                    rows_v.at[b], acc.at[dst_v.at[j]], ssems[b]).wait()

            for b in range(_GAHEAD):
                g_start(b, b)

            def body(i, carry):
                j0 = i * _NBUF
                for t in range(_NBUF):
                    j = j0 + t
                    g_wait(j, t)
                    pltpu.sync_copy(rows_v.at[t], acc.at[dst_v.at[j]], add=True)

                    @pl.when(j + _NBUF < _NCHUNK)
                    def _(j=j, t=t):
                        g_start(j + _NBUF, t)

                return carry

            lax.fori_loop(0, _NCHUNK // _NBUF, body, 0)

        @pl.when(c == 0)
        def _():
            run(hL_hbm)

        @pl.when(c == 1)
        def _():
            run(hR_hbm)

        plsc.subcore_barrier()
        pltpu.sync_copy(acc.at[pl.ds(s * _RPT, _RPT)],
                        out_hbm.at[c, pl.ds(s * _RPT, _RPT)])

    return k(hL, hR, eidx, zeros)


_BR = 1000


def _gin_dense(h, p, eps, W, b, Rw, Rb):
    def body(eps_ref, h_ref, p_ref, W_ref, b_ref, Rw_ref, Rb_ref,
             o_ref, oL_ref, oR_ref):
        agg = jnp.concatenate([p_ref[0], p_ref[1]], axis=1)
        x = (1.0 + eps_ref[0]) * h_ref[...] + agg
        y = jnp.dot(x, W_ref[...], preferred_element_type=jnp.float32) + b_ref[...]
        z = jnp.maximum(y, 0.0)
        w = jnp.dot(z, Rw_ref[...], preferred_element_type=jnp.float32) + Rb_ref[...]
        res = z + jnp.maximum(w, 0.0)
        o_ref[...] = res
        oL_ref[...] = res[:, :_HD]
        oR_ref[...] = res[:, _HD:]

    return pl.pallas_call(
        body,
        grid=(N // _BR,),
        in_specs=[
            pl.BlockSpec(memory_space=pltpu.SMEM),
            pl.BlockSpec((_BR, D), lambda i: (i, 0)),
            pl.BlockSpec((_NC, _BR, _HD), lambda i: (0, i, 0)),
            pl.BlockSpec((D, D), lambda i: (0, 0)),
            pl.BlockSpec((1, D), lambda i: (0, 0)),
            pl.BlockSpec((D, D), lambda i: (0, 0)),
            pl.BlockSpec((1, D), lambda i: (0, 0)),
        ],
        out_specs=[
            pl.BlockSpec((_BR, D), lambda i: (i, 0)),
            pl.BlockSpec((_BR, _HD), lambda i: (i, 0)),
            pl.BlockSpec((_BR, _HD), lambda i: (i, 0)),
        ],
        out_shape=[
            jax.ShapeDtypeStruct((N, D), jnp.float32),
            jax.ShapeDtypeStruct((N, _HD), jnp.float32),
            jax.ShapeDtypeStruct((N, _HD), jnp.float32),
        ],
        compiler_params=pltpu.CompilerParams(dimension_semantics=("parallel",)),
    )(eps.reshape(1), h, p, W, b.reshape(1, D), Rw, Rb.reshape(1, D))


def _final(h1, p, eps, W, b, Rw, Rb, Lw, Lb):
    def body(eps_ref, h_ref, p_ref, W_ref, b_ref, Rw_ref, Rb_ref,
             Lw_ref, Lb_ref, o_ref):
        h1b = h_ref[...]
        agg = jnp.concatenate([p_ref[0], p_ref[1]], axis=1)
        x = (1.0 + eps_ref[0]) * h1b + agg
        y = jnp.dot(x, W_ref[...], preferred_element_type=jnp.float32) + b_ref[...]
        z = jnp.maximum(y, 0.0)
        w = jnp.dot(z, Rw_ref[...], preferred_element_type=jnp.float32) + Rb_ref[...]
        h2b = z + jnp.maximum(w, 0.0)
        m = 0.5 * (h1b + h2b)
        o_ref[...] = jnp.dot(m, Lw_ref[...], preferred_element_type=jnp.float32) + Lb_ref[...]

    return pl.pallas_call(
        body,
        grid=(N // _BR,),
        in_specs=[
            pl.BlockSpec(memory_space=pltpu.SMEM),
            pl.BlockSpec((_BR, D), lambda i: (i, 0)),
            pl.BlockSpec((_NC, _BR, _HD), lambda i: (0, i, 0)),
            pl.BlockSpec((D, D), lambda i: (0, 0)),
            pl.BlockSpec((1, D), lambda i: (0, 0)),
            pl.BlockSpec((D, D), lambda i: (0, 0)),
            pl.BlockSpec((1, D), lambda i: (0, 0)),
            pl.BlockSpec((D, L), lambda i: (0, 0)),
            pl.BlockSpec((1, L), lambda i: (0, 0)),
        ],
        out_specs=pl.BlockSpec((_BR, L), lambda i: (i, 0)),
        out_shape=jax.ShapeDtypeStruct((N, L), jnp.float32),
        compiler_params=pltpu.CompilerParams(dimension_semantics=("parallel",)),
    )(eps.reshape(1), h1, p, W, b.reshape(1, D), Rw, Rb.reshape(1, D),
      Lw, Lb.reshape(1, L))


def kernel(inputs, edge_index, eps1, W1, b1, eps2, W2, b2, Rw1, Rb1, Rw2, Rb2, Lw, Lb):
    eidx = edge_index.reshape(2, E // _CH, _CH)
    zeros = jnp.zeros((_RPT, _HD), jnp.float32)
    p1 = _sc_scatter_add(inputs[:, :_HD], inputs[:, _HD:], eidx, zeros)
    h1, h1L, h1R = _gin_dense(inputs, p1, eps1, W1, b1, Rw1, Rb1)
    p2 = _sc_scatter_add(h1L, h1R, eidx, zeros)
    return _final(h1, p2, eps2, W2, b2, Rw2, Rb2, Lw, Lb)


# speedup vs baseline: 1.2413x; 1.0003x over previous
import functools

import jax
import jax.numpy as jnp
from jax import lax
from jax.experimental import pallas as pl
from jax.experimental.pallas import tpu as pltpu
from jax.experimental.pallas import tpu_sc as plsc

N = 10000
E = 320000
D = 128
L = 4096

_NC = 2
_NS = 16
_HD = D // _NC
_CH = 125
_EPT = E // _NS
_NCHUNK = _EPT // _CH
_NBUF = 5
_GAHEAD = 5
_RPT = 640
_NPAD = _RPT * _NS


def _sc_scatter_add(hL, hR, eidx, zeros):
    mesh = plsc.VectorSubcoreMesh(core_axis_name="c", subcore_axis_name="s")

    @functools.partial(
        pl.kernel,
        mesh=mesh,
        compiler_params=pltpu.CompilerParams(use_tc_tiling_on_sc=False),
        out_type=jax.ShapeDtypeStruct((_NC, _NPAD, _HD), jnp.float32),
        scratch_types=[
            pltpu.VMEM((_NCHUNK, _CH), jnp.int32),
            pltpu.VMEM((_NCHUNK, _CH), jnp.int32),
            pltpu.VMEM((_NBUF, _CH, _HD), jnp.float32),
            pltpu.VMEM_SHARED((_NPAD, _HD), jnp.float32),
            [pltpu.SemaphoreType.DMA] * _NBUF,
        ],
    )
    def k(hL_hbm, hR_hbm, eidx_hbm, z_hbm, out_hbm,
          src_v, dst_v, rows_v, acc, gsems):
        c = lax.axis_index("c")
        s = lax.axis_index("s")
        pltpu.sync_copy(z_hbm, acc.at[pl.ds(s * _RPT, _RPT)])
        pltpu.sync_copy(eidx_hbm.at[0, pl.ds(s * _NCHUNK, _NCHUNK)], src_v)
        pltpu.sync_copy(eidx_hbm.at[1, pl.ds(s * _NCHUNK, _NCHUNK)], dst_v)
        plsc.subcore_barrier()

        def run(h_hbm):
            def g_start(j, b):
                pltpu.make_async_copy(
                    h_hbm.at[src_v.at[j]], rows_v.at[b], gsems[b]).start()

            def g_wait(j, b):
                pltpu.make_async_copy(
                    h_hbm.at[src_v.at[j]], rows_v.at[b], gsems[b]).wait()

            for b in range(_GAHEAD):
                g_start(b, b)

            def body(i, carry):
                j0 = i * _NBUF
                for t in range(_NBUF):
                    j = j0 + t
                    g_wait(j, t)
                    pltpu.sync_copy(rows_v.at[t], acc.at[dst_v.at[j]], add=True)

                    @pl.when(j + _NBUF < _NCHUNK)
                    def _(j=j, t=t):
                        g_start(j + _NBUF, t)

                return carry

            lax.fori_loop(0, _NCHUNK // _NBUF, body, 0)

        @pl.when(c == 0)
        def _():
            run(hL_hbm)

        @pl.when(c == 1)
        def _():
            run(hR_hbm)

        plsc.subcore_barrier()
        pltpu.sync_copy(acc.at[pl.ds(s * _RPT, _RPT)],
                        out_hbm.at[c, pl.ds(s * _RPT, _RPT)])

    return k(hL, hR, eidx, zeros)


_BR = 1000


def _gin_dense(h, p, eps, W, b, Rw, Rb):
    def body(eps_ref, h_ref, p_ref, W_ref, b_ref, Rw_ref, Rb_ref,
             o_ref, oL_ref, oR_ref):
        agg = jnp.concatenate([p_ref[0], p_ref[1]], axis=1)
        x = (1.0 + eps_ref[0]) * h_ref[...] + agg
        y = jnp.dot(x, W_ref[...], preferred_element_type=jnp.float32) + b_ref[...]
        z = jnp.maximum(y, 0.0)
        w = jnp.dot(z, Rw_ref[...], preferred_element_type=jnp.float32) + Rb_ref[...]
        res = z + jnp.maximum(w, 0.0)
        o_ref[...] = res
        oL_ref[...] = res[:, :_HD]
        oR_ref[...] = res[:, _HD:]

    return pl.pallas_call(
        body,
        grid=(N // _BR,),
        in_specs=[
            pl.BlockSpec(memory_space=pltpu.SMEM),
            pl.BlockSpec((_BR, D), lambda i: (i, 0)),
            pl.BlockSpec((_NC, _BR, _HD), lambda i: (0, i, 0)),
            pl.BlockSpec((D, D), lambda i: (0, 0)),
            pl.BlockSpec((1, D), lambda i: (0, 0)),
            pl.BlockSpec((D, D), lambda i: (0, 0)),
            pl.BlockSpec((1, D), lambda i: (0, 0)),
        ],
        out_specs=[
            pl.BlockSpec((_BR, D), lambda i: (i, 0)),
            pl.BlockSpec((_BR, _HD), lambda i: (i, 0)),
            pl.BlockSpec((_BR, _HD), lambda i: (i, 0)),
        ],
        out_shape=[
            jax.ShapeDtypeStruct((N, D), jnp.float32),
            jax.ShapeDtypeStruct((N, _HD), jnp.float32),
            jax.ShapeDtypeStruct((N, _HD), jnp.float32),
        ],
        compiler_params=pltpu.CompilerParams(dimension_semantics=("parallel",)),
    )(eps.reshape(1), h, p, W, b.reshape(1, D), Rw, Rb.reshape(1, D))


def _final(h1, p, eps, W, b, Rw, Rb, Lw, Lb):
    def body(eps_ref, h_ref, p_ref, W_ref, b_ref, Rw_ref, Rb_ref,
             Lw_ref, Lb_ref, o_ref):
        h1b = h_ref[...]
        agg = jnp.concatenate([p_ref[0], p_ref[1]], axis=1)
        x = (1.0 + eps_ref[0]) * h1b + agg
        y = jnp.dot(x, W_ref[...], preferred_element_type=jnp.float32) + b_ref[...]
        z = jnp.maximum(y, 0.0)
        w = jnp.dot(z, Rw_ref[...], preferred_element_type=jnp.float32) + Rb_ref[...]
        h2b = z + jnp.maximum(w, 0.0)
        m = 0.5 * (h1b + h2b)
        o_ref[...] = jnp.dot(m, Lw_ref[...], preferred_element_type=jnp.float32) + Lb_ref[...]

    return pl.pallas_call(
        body,
        grid=(N // _BR,),
        in_specs=[
            pl.BlockSpec(memory_space=pltpu.SMEM),
            pl.BlockSpec((_BR, D), lambda i: (i, 0)),
            pl.BlockSpec((_NC, _BR, _HD), lambda i: (0, i, 0)),
            pl.BlockSpec((D, D), lambda i: (0, 0)),
            pl.BlockSpec((1, D), lambda i: (0, 0)),
            pl.BlockSpec((D, D), lambda i: (0, 0)),
            pl.BlockSpec((1, D), lambda i: (0, 0)),
            pl.BlockSpec((D, L), lambda i: (0, 0)),
            pl.BlockSpec((1, L), lambda i: (0, 0)),
        ],
        out_specs=pl.BlockSpec((_BR, L), lambda i: (i, 0)),
        out_shape=jax.ShapeDtypeStruct((N, L), jnp.float32),
        compiler_params=pltpu.CompilerParams(dimension_semantics=("parallel",)),
    )(eps.reshape(1), h1, p, W, b.reshape(1, D), Rw, Rb.reshape(1, D),
      Lw, Lb.reshape(1, L))


def kernel(inputs, edge_index, eps1, W1, b1, eps2, W2, b2, Rw1, Rb1, Rw2, Rb2, Lw, Lb):
    eidx = edge_index.reshape(2, E // _CH, _CH)
    zeros = jnp.zeros((_RPT, _HD), jnp.float32)
    p1 = _sc_scatter_add(inputs[:, :_HD], inputs[:, _HD:], eidx, zeros)
    h1, h1L, h1R = _gin_dense(inputs, p1, eps1, W1, b1, Rw1, Rb1)
    p2 = _sc_scatter_add(h1L, h1R, eidx, zeros)
    return _final(h1, p2, eps2, W2, b2, Rw2, Rb2, Lw, Lb)
